# baseline (speedup 1.0000x reference)
import functools

import jax
import jax.numpy as jnp
from jax import lax
from jax.experimental import pallas as pl
from jax.experimental.pallas import tpu as pltpu
from jax.experimental.pallas import tpu_sc as plsc

L_LAYERS = 16
SEQ = 4096
E = 64
NC = 2
NS = 16
LANES = 16
NPAIR = E // 2

TOK_PER_TEC = SEQ // NC
CHUNK = 1024
ROWS = CHUNK // 4
N_CHUNKS = TOK_PER_TEC // CHUNK
SUPER = ROWS // LANES
N_CHAINS = 8

_mesh = plsc.VectorSubcoreMesh(
    core_axis_name="c", subcore_axis_name="s", num_cores=NC, num_subcores=NS
)


@functools.partial(
    pl.kernel,
    out_type=jax.ShapeDtypeStruct((NS, NC, E * LANES), jnp.float32),
    mesh=_mesh,
    scratch_types=[
        pltpu.VMEM((ROWS, 129), jnp.float32),
        pltpu.VMEM((E * LANES,), jnp.float32),
    ],
    compiler_params=pltpu.CompilerParams(
        use_tc_tiling_on_sc=False, needs_layout_passes=False
    ),
)
def _sc_count(w_hbm, cnt_out, chunk_vm, cnt_vm):
    c = lax.axis_index("c")
    s = lax.axis_index("s")

    iota = lax.iota(jnp.int32, LANES)
    zero = jnp.zeros((LANES,), jnp.float32)
    ones = jnp.full((LANES,), 1.0, jnp.float32)
    neg1 = jnp.full((LANES,), -1, jnp.int32)
    lomask = jnp.full((LANES,), 63, jnp.int32)
    col_splats = [jnp.full((LANES,), col, jnp.int32) for col in range(2 * E)]
    id_splats = [jnp.full((LANES,), e, jnp.int32) for e in range(E)]

    for k in range(E):
        cnt_vm[pl.ds(k * LANES, LANES)] = zero

    def merge(a, b):
        k1a, k2a = a
        k1b, k2b = b
        return (
            jnp.maximum(k1a, k1b),
            jnp.maximum(jnp.minimum(k1a, k1b), jnp.maximum(k2a, k2b)),
        )

    row_base = s * (SEQ // 4) + c * (TOK_PER_TEC // 4)
    for ci in range(N_CHUNKS):
        r0 = pl.multiple_of(row_base + ci * ROWS, ROWS)
        pltpu.sync_copy(
            w_hbm.at[pl.ds(r0, ROWS), :],
            chunk_vm.at[:, pl.ds(0, 2 * E)],
        )

        @plsc.parallel_loop(0, SUPER, 1, unroll=2)
        def _sweep(w):
            row = w * LANES + iota
            for b in range(4):
                k1s = [neg1] * N_CHAINS
                k2s = [neg1] * N_CHAINS
                for k in range(NPAIR):
                    j = k % N_CHAINS
                    col = b * NPAIR + k
                    v = plsc.load_gather(chunk_vm, [row, col_splats[col]])
                    vb = plsc.bitcast(v, jnp.bfloat16)
                    lo, hi = plsc.unpack(
                        vb,
                        format=plsc.PackFormat.INTERLEAVED,
                        preferred_element_type=jnp.float32,
                    )
                    key_lo = plsc.bitcast(lo, jnp.int32) | id_splats[2 * k]
                    key_hi = plsc.bitcast(hi, jnp.int32) | id_splats[2 * k + 1]
                    k2s[j] = jnp.maximum(k2s[j], jnp.minimum(k1s[j], key_lo))
                    k1s[j] = jnp.maximum(k1s[j], key_lo)
                    k2s[j] = jnp.maximum(k2s[j], jnp.minimum(k1s[j], key_hi))
                    k1s[j] = jnp.maximum(k1s[j], key_hi)
                ps = list(zip(k1s, k2s))
                while len(ps) > 1:
                    ps = [merge(ps[i], ps[i + 1]) for i in range(0, len(ps), 2)]
                k1, k2 = ps[0]
                idx1 = (k1 & lomask) * LANES + iota
                idx2 = (k2 & lomask) * LANES + iota
                plsc.addupdate_scatter(cnt_vm, [idx1], ones)
                plsc.addupdate_scatter(cnt_vm, [idx2], ones)

    pltpu.sync_copy(cnt_vm, cnt_out.at[s, c])


def _gsum_body(w_ref, out_ref):
    h = pl.program_id(1)

    @pl.when(h == 0)
    def _init():
        out_ref[...] = jnp.zeros_like(out_ref)

    x = w_ref[...]
    xi = lax.bitcast_convert_type(x, jnp.int32)
    lo = lax.bitcast_convert_type(xi << 16, jnp.float32)
    hi = lax.bitcast_convert_type(xi & jnp.int32(-65536), jnp.float32)
    s_lo = jnp.sum(lo, axis=0)
    s_hi = jnp.sum(hi, axis=0)
    out_ref[...] += jnp.concatenate([s_lo, s_hi]).reshape(1, 1, 4 * E)


def _combine_body(scale_ref, cnt_ref, gsum_ref, out_ref):
    x = cnt_ref[...]
    x1 = x[:, : E * LANES] + x[:, E * LANES :]
    i = lax.broadcasted_iota(jnp.int32, (E * LANES, E), 0)
    j = lax.broadcasted_iota(jnp.int32, (E * LANES, E), 1)
    perm_j = 2 * (j & 31) + (j >> 5)
    sel = jnp.where(i // LANES == perm_j, 1.0, 0.0)
    cs = jnp.dot(x1, sel, preferred_element_type=jnp.float32)
    g = gsum_ref[...]
    glo = g[:, : 2 * E]
    ghi = g[:, 2 * E :]
    rlo = glo[:, :32] + glo[:, 32:64] + glo[:, 64:96] + glo[:, 96:128]
    rhi = ghi[:, :32] + ghi[:, 32:64] + ghi[:, 64:96] + ghi[:, 96:128]
    gperm = jnp.concatenate([rlo, rhi], axis=1)
    out_ref[0, 0] = jnp.sum(cs * gperm) * scale_ref[0]


def kernel(router_weights, router_logits, num_experts_per_tok, non_pad_token):
    del router_logits
    wb = router_weights.astype(jnp.bfloat16)
    wq = lax.bitcast_convert_type(
        wb.reshape(L_LAYERS, SEQ, NPAIR, 2), jnp.float32
    )
    wq = lax.optimization_barrier(wq.reshape(L_LAYERS * SEQ // 4, 2 * E))
    cnt_p = _sc_count(wq)
    gsum = pl.pallas_call(
        _gsum_body,
        grid=(L_LAYERS, NC),
        in_specs=[
            pl.BlockSpec((SEQ // 8, 2 * E), lambda l, h: (l * NC + h, 0)),
        ],
        out_specs=pl.BlockSpec((1, 1, 4 * E), lambda l, h: (l, 0, 0)),
        out_shape=jax.ShapeDtypeStruct((L_LAYERS, 1, 4 * E), jnp.float32),
    )(wq)
    gsum = gsum.reshape(L_LAYERS, 4 * E)
    valid = jnp.maximum(non_pad_token, 1)
    scale = (E / (valid * num_experts_per_tok)) / valid
    scale = jnp.asarray(scale, jnp.float32).reshape((1,))
    cnt2 = cnt_p.reshape(L_LAYERS, NC * E * LANES)
    out = pl.pallas_call(
        _combine_body,
        out_shape=jax.ShapeDtypeStruct((1, 1), jnp.float32),
        in_specs=[
            pl.BlockSpec(memory_space=pltpu.SMEM),
            pl.BlockSpec(memory_space=pltpu.VMEM),
            pl.BlockSpec(memory_space=pltpu.VMEM),
        ],
        out_specs=pl.BlockSpec(memory_space=pltpu.SMEM),
    )(scale, cnt2, gsum)
    return out[0, 0]


# speedup vs baseline: 1.5707x; 1.5707x over previous
import functools

import jax
import jax.numpy as jnp
from jax import lax
from jax.experimental import pallas as pl
from jax.experimental.pallas import tpu as pltpu
from jax.experimental.pallas import tpu_sc as plsc

L_LAYERS = 16
SEQ = 4096
E = 64
NC = 2
NS = 16
LANES = 16

TOK_PER_TEC = SEQ // NC
CHUNK = 1024
GROUPS = CHUNK // LANES
N_CHUNKS = TOK_PER_TEC // CHUNK
N_CHAINS = 8

_mesh = plsc.VectorSubcoreMesh(
    core_axis_name="c", subcore_axis_name="s", num_cores=NC, num_subcores=NS
)


@functools.partial(
    pl.kernel,
    out_type=jax.ShapeDtypeStruct((NS, NC, E * LANES), jnp.float32),
    mesh=_mesh,
    scratch_types=[
        pltpu.VMEM((CHUNK // 2, 129), jnp.float32),
        pltpu.VMEM((E * LANES,), jnp.float32),
    ],
    compiler_params=pltpu.CompilerParams(
        use_tc_tiling_on_sc=False, needs_layout_passes=False
    ),
)
def _sc_count(w_hbm, cnt_out, chunk_vm, cnt_vm):
    c = lax.axis_index("c")
    s = lax.axis_index("s")

    iota = lax.iota(jnp.int32, LANES)
    zero = jnp.zeros((LANES,), jnp.float32)
    ones = jnp.full((LANES,), 1.0, jnp.float32)
    neg1 = jnp.full((LANES,), -1, jnp.int32)
    himask = jnp.full((LANES,), ~127, jnp.int32)
    lomask = jnp.full((LANES,), 63, jnp.int32)
    col_splats = [jnp.full((LANES,), col, jnp.int32) for col in range(2 * E)]

    for k in range(E):
        cnt_vm[pl.ds(k * LANES, LANES)] = zero

    def merge(a, b):
        k1a, k2a = a
        k1b, k2b = b
        return (
            jnp.maximum(k1a, k1b),
            jnp.maximum(jnp.minimum(k1a, k1b), jnp.maximum(k2a, k2b)),
        )

    row_base = s * (SEQ // 2) + c * (TOK_PER_TEC // 2)
    for ci in range(N_CHUNKS):
        r0 = pl.multiple_of(row_base + ci * (CHUNK // 2), CHUNK // 2)
        pltpu.sync_copy(
            w_hbm.at[pl.ds(r0, CHUNK // 2), :],
            chunk_vm.at[:, pl.ds(0, 2 * E)],
        )

        @plsc.parallel_loop(0, GROUPS // 2, 1, unroll=2)
        def _sweep(w):
            row = w * LANES + iota
            for parity in range(2):
                k1s = [neg1] * N_CHAINS
                k2s = [neg1] * N_CHAINS
                for e in range(E):
                    j = e % N_CHAINS
                    col = parity * E + e
                    v = plsc.load_gather(chunk_vm, [row, col_splats[col]])
                    key = (plsc.bitcast(v, jnp.int32) & himask) | col_splats[col]
                    k2s[j] = jnp.maximum(k2s[j], jnp.minimum(k1s[j], key))
                    k1s[j] = jnp.maximum(k1s[j], key)
                ps = list(zip(k1s, k2s))
                while len(ps) > 1:
                    ps = [merge(ps[i], ps[i + 1]) for i in range(0, len(ps), 2)]
                k1, k2 = ps[0]
                idx1 = (k1 & lomask) * LANES + iota
                idx2 = (k2 & lomask) * LANES + iota
                plsc.addupdate_scatter(cnt_vm, [idx1], ones)
                plsc.addupdate_scatter(cnt_vm, [idx2], ones)

    pltpu.sync_copy(cnt_vm, cnt_out.at[s, c])


def _gsum_body(w_ref, out_ref):
    h = pl.program_id(1)

    @pl.when(h == 0)
    def _init():
        out_ref[...] = jnp.zeros_like(out_ref)

    out_ref[...] += jnp.sum(w_ref[...], axis=0)[None, None, :]


def _combine_body(scale_ref, cnt_ref, gsum_ref, out_ref):
    x = cnt_ref[...]
    x1 = x[:, : E * LANES] + x[:, E * LANES :]
    i = lax.broadcasted_iota(jnp.int32, (E * LANES, E), 0)
    j = lax.broadcasted_iota(jnp.int32, (E * LANES, E), 1)
    sel = jnp.where(i // LANES == j, 1.0, 0.0)
    cs = jnp.dot(x1, sel, preferred_element_type=jnp.float32)
    g = gsum_ref[...]
    gs = g[:, :E] + g[:, E:]
    out_ref[0, 0] = jnp.sum(cs * gs) * scale_ref[0]


_TB = 1024


def kernel(router_weights, router_logits, num_experts_per_tok, non_pad_token):
    del router_logits
    w2 = lax.optimization_barrier(
        router_weights.reshape(L_LAYERS * SEQ // 2, 2 * E)
    )
    cnt_p = _sc_count(w2)
    gsum = pl.pallas_call(
        _gsum_body,
        grid=(L_LAYERS, NC),
        in_specs=[
            pl.BlockSpec((SEQ // 4, 2 * E), lambda l, h: (l * NC + h, 0)),
        ],
        out_specs=pl.BlockSpec((1, 1, 2 * E), lambda l, h: (l, 0, 0)),
        out_shape=jax.ShapeDtypeStruct((L_LAYERS, 1, 2 * E), jnp.float32),
    )(w2)
    gsum = gsum.reshape(L_LAYERS, 2 * E)
    valid = jnp.maximum(non_pad_token, 1)
    scale = (E / (valid * num_experts_per_tok)) / valid
    scale = jnp.asarray(scale, jnp.float32).reshape((1,))
    cnt2 = cnt_p.reshape(L_LAYERS, NC * E * LANES)
    out = pl.pallas_call(
        _combine_body,
        out_shape=jax.ShapeDtypeStruct((1, 1), jnp.float32),
        in_specs=[
            pl.BlockSpec(memory_space=pltpu.SMEM),
            pl.BlockSpec(memory_space=pltpu.VMEM),
            pl.BlockSpec(memory_space=pltpu.VMEM),
        ],
        out_specs=pl.BlockSpec(memory_space=pltpu.SMEM),
    )(scale, cnt2, gsum)
    return out[0, 0]


# speedup vs baseline: 1.6030x; 1.0206x over previous
import functools

import jax
import jax.numpy as jnp
from jax import lax
from jax.experimental import pallas as pl
from jax.experimental.pallas import tpu as pltpu
from jax.experimental.pallas import tpu_sc as plsc

L_LAYERS = 16
SEQ = 4096
E = 64
NC = 2
NS = 16
LANES = 16

TOK_PER_TEC = SEQ // NC
CHUNK = 512
GROUPS = CHUNK // LANES
N_CHUNKS = TOK_PER_TEC // CHUNK
N_CHAINS = 8

_mesh = plsc.VectorSubcoreMesh(
    core_axis_name="c", subcore_axis_name="s", num_cores=NC, num_subcores=NS
)


@functools.partial(
    pl.kernel,
    out_type=jax.ShapeDtypeStruct((NS, NC, E * LANES), jnp.float32),
    mesh=_mesh,
    scratch_types=[
        pltpu.VMEM((CHUNK // 2, 129), jnp.float32),
        pltpu.VMEM((CHUNK // 2, 129), jnp.float32),
        pltpu.VMEM((E * LANES,), jnp.float32),
        pltpu.SemaphoreType.DMA,
        pltpu.SemaphoreType.DMA,
    ],
    compiler_params=pltpu.CompilerParams(
        use_tc_tiling_on_sc=False, needs_layout_passes=False
    ),
)
def _sc_count(w_hbm, cnt_out, chunk_a, chunk_b, cnt_vm, sem_a, sem_b):
    c = lax.axis_index("c")
    s = lax.axis_index("s")

    iota = lax.iota(jnp.int32, LANES)
    zero = jnp.zeros((LANES,), jnp.float32)
    ones = jnp.full((LANES,), 1.0, jnp.float32)
    neg1 = jnp.full((LANES,), -1, jnp.int32)
    himask = jnp.full((LANES,), ~127, jnp.int32)
    lomask = jnp.full((LANES,), 63, jnp.int32)
    col_splats = [jnp.full((LANES,), col, jnp.int32) for col in range(2 * E)]

    for k in range(E):
        cnt_vm[pl.ds(k * LANES, LANES)] = zero

    def merge(a, b):
        k1a, k2a = a
        k1b, k2b = b
        return (
            jnp.maximum(k1a, k1b),
            jnp.maximum(jnp.minimum(k1a, k1b), jnp.maximum(k2a, k2b)),
        )

    row_base = s * (SEQ // 2) + c * (TOK_PER_TEC // 2)
    bufs = [chunk_a, chunk_b]
    sems = [sem_a, sem_b]

    def start(ci):
        r0 = pl.multiple_of(row_base + ci * (CHUNK // 2), CHUNK // 2)
        return pltpu.async_copy(
            w_hbm.at[pl.ds(r0, CHUNK // 2), :],
            bufs[ci % 2].at[:, pl.ds(0, 2 * E)],
            sems[ci % 2],
        )

    pending = start(0)
    for ci in range(N_CHUNKS):
        nxt = start(ci + 1) if ci + 1 < N_CHUNKS else None
        pending.wait()
        pending = nxt
        chunk_vm = bufs[ci % 2]

        @plsc.parallel_loop(0, GROUPS // 2, 1, unroll=2)
        def _sweep(w):
            row = w * LANES + iota
            for parity in range(2):
                k1s = [neg1] * N_CHAINS
                k2s = [neg1] * N_CHAINS
                for e in range(E):
                    j = e % N_CHAINS
                    col = parity * E + e
                    v = plsc.load_gather(chunk_vm, [row, col_splats[col]])
                    key = (plsc.bitcast(v, jnp.int32) & himask) | col_splats[col]
                    k2s[j] = jnp.maximum(k2s[j], jnp.minimum(k1s[j], key))
                    k1s[j] = jnp.maximum(k1s[j], key)
                ps = list(zip(k1s, k2s))
                while len(ps) > 1:
                    ps = [merge(ps[i], ps[i + 1]) for i in range(0, len(ps), 2)]
                k1, k2 = ps[0]
                idx1 = (k1 & lomask) * LANES + iota
                idx2 = (k2 & lomask) * LANES + iota
                plsc.addupdate_scatter(cnt_vm, [idx1], ones)
                plsc.addupdate_scatter(cnt_vm, [idx2], ones)

    pltpu.sync_copy(cnt_vm, cnt_out.at[s, c])


def _gsum_body(w_ref, out_ref):
    h = pl.program_id(1)

    @pl.when(h == 0)
    def _init():
        out_ref[...] = jnp.zeros_like(out_ref)

    out_ref[...] += jnp.sum(w_ref[...], axis=0)[None, None, :]


def _combine_body(scale_ref, cnt_ref, gsum_ref, out_ref):
    x = cnt_ref[...]
    x1 = x[:, : E * LANES] + x[:, E * LANES :]
    i = lax.broadcasted_iota(jnp.int32, (E * LANES, E), 0)
    j = lax.broadcasted_iota(jnp.int32, (E * LANES, E), 1)
    sel = jnp.where(i // LANES == j, 1.0, 0.0)
    cs = jnp.dot(x1, sel, preferred_element_type=jnp.float32)
    g = gsum_ref[...]
    gs = g[:, :E] + g[:, E:]
    out_ref[0, 0] = jnp.sum(cs * gs) * scale_ref[0]


_TB = 1024


def kernel(router_weights, router_logits, num_experts_per_tok, non_pad_token):
    del router_logits
    w2 = lax.optimization_barrier(
        router_weights.reshape(L_LAYERS * SEQ // 2, 2 * E)
    )
    cnt_p = _sc_count(w2)
    gsum = pl.pallas_call(
        _gsum_body,
        grid=(L_LAYERS, NC),
        in_specs=[
            pl.BlockSpec((SEQ // 4, 2 * E), lambda l, h: (l * NC + h, 0)),
        ],
        out_specs=pl.BlockSpec((1, 1, 2 * E), lambda l, h: (l, 0, 0)),
        out_shape=jax.ShapeDtypeStruct((L_LAYERS, 1, 2 * E), jnp.float32),
    )(w2)
    gsum = gsum.reshape(L_LAYERS, 2 * E)
    valid = jnp.maximum(non_pad_token, 1)
    scale = (E / (valid * num_experts_per_tok)) / valid
    scale = jnp.asarray(scale, jnp.float32).reshape((1,))
    cnt2 = cnt_p.reshape(L_LAYERS, NC * E * LANES)
    out = pl.pallas_call(
        _combine_body,
        out_shape=jax.ShapeDtypeStruct((1, 1), jnp.float32),
        in_specs=[
            pl.BlockSpec(memory_space=pltpu.SMEM),
            pl.BlockSpec(memory_space=pltpu.VMEM),
            pl.BlockSpec(memory_space=pltpu.VMEM),
        ],
        out_specs=pl.BlockSpec(memory_space=pltpu.SMEM),
    )(scale, cnt2, gsum)
    return out[0, 0]
